# Initial kernel scaffold; baseline (speedup 1.0000x reference)
import jax
import jax.numpy as jnp
from jax.experimental import pallas as pl


def kernel(node_feature, edge_index, node_id_index, weight, weight_id):
    raise NotImplementedError("write your pallas kernel here")


# speedup vs baseline: 7.8295x; 7.8295x over previous
import functools

import jax
import jax.numpy as jnp
from jax import lax
from jax.experimental import pallas as pl
from jax.experimental.pallas import tpu as pltpu
from jax.experimental.pallas import tpu_sc as plsc

N_NODES = 10000
N_EDGES = 320000
D = 128
DH = 64
CW = 16
NUM_IDX = 2000

NC = 2
NS = 16
NW = NC * NS
K = 80
CH_PER_TILE = N_EDGES // (NS * K)
N_PAD = 10240
ROWS_PER_TILE = N_PAD // NS
ZROWS = 128
NID_K = 80
NID_CHUNKS = NUM_IDX // NID_K


def _sc_body(xh_hbm, src_hbm, dst_hbm, nid_hbm, ones_hbm, idsrc_hbm,
             feat_hbm, cnt_hbm,
             srcv, dstv, bufa, bufb, onesv, idsrcv, zbuf, zcnt, nidv,
             acc, cac, sema, semb):
    c = lax.axis_index("c")
    s = lax.axis_index("s")

    def zrow(i, carry):
        for jj in range(DH // 16):
            zbuf[i, pl.ds(jj * 16, 16)] = jnp.zeros((16,), jnp.float32)
        zcnt[i, pl.ds(0, 16)] = jnp.zeros((16,), jnp.float32)
        return carry

    lax.fori_loop(0, ZROWS, zrow, 0)
    for r in range(ROWS_PER_TILE // ZROWS):
        base = s * ROWS_PER_TILE + r * ZROWS
        pltpu.sync_copy(zbuf, acc.at[pl.ds(base, ZROWS)])
        pltpu.sync_copy(zcnt, cac.at[pl.ds(base, ZROWS)])
    plsc.subcore_barrier()

    pltpu.sync_copy(src_hbm.at[c, s], srcv)
    pltpu.sync_copy(dst_hbm.at[s], dstv)
    pltpu.sync_copy(ones_hbm, onesv)

    pltpu.async_copy(xh_hbm.at[srcv.at[0]], bufa, sema)
    pltpu.async_copy(xh_hbm.at[srcv.at[1]], bufb, semb)

    def body(it, carry):
        j = it * 2
        pltpu.make_async_copy(xh_hbm.at[srcv.at[j]], bufa, sema).wait()
        pltpu.sync_copy(bufa, acc.at[dstv.at[j]], add=True)
        pltpu.sync_copy(onesv, cac.at[dstv.at[j]], add=True)

        @pl.when(j + 2 < CH_PER_TILE)
        def _():
            pltpu.async_copy(xh_hbm.at[srcv.at[j + 2]], bufa, sema)

        pltpu.make_async_copy(xh_hbm.at[srcv.at[j + 1]], bufb, semb).wait()
        pltpu.sync_copy(bufb, acc.at[dstv.at[j + 1]], add=True)
        pltpu.sync_copy(onesv, cac.at[dstv.at[j + 1]], add=True)

        @pl.when(j + 3 < CH_PER_TILE)
        def _():
            pltpu.async_copy(xh_hbm.at[srcv.at[j + 3]], bufb, semb)

        return carry

    lax.fori_loop(0, CH_PER_TILE // 2, body, 0)

    @pl.when((c == 0) & (s == 0))
    def _():
        pltpu.sync_copy(nid_hbm, nidv)
        pltpu.sync_copy(idsrc_hbm, idsrcv)

        def idbody(j, carry):
            pltpu.sync_copy(idsrcv, cac.at[nidv.at[j]], add=True)
            return carry

        lax.fori_loop(0, NID_CHUNKS, idbody, 0)

    plsc.subcore_barrier()

    rb = s * ROWS_PER_TILE
    pltpu.sync_copy(acc.at[pl.ds(rb, ROWS_PER_TILE)],
                    feat_hbm.at[c, pl.ds(rb, ROWS_PER_TILE)])
    pltpu.sync_copy(cac.at[pl.ds(rb, ROWS_PER_TILE)],
                    cnt_hbm.at[c, pl.ds(rb, ROWS_PER_TILE)])


_sc_call = functools.partial(
    pl.kernel,
    out_type=(jax.ShapeDtypeStruct((NC, N_PAD, DH), jnp.float32),
              jax.ShapeDtypeStruct((NC, N_PAD, CW), jnp.float32)),
    mesh=plsc.VectorSubcoreMesh(core_axis_name="c", subcore_axis_name="s"),
    compiler_params=pltpu.CompilerParams(use_tc_tiling_on_sc=False),
    scratch_types=[
        pltpu.VMEM((CH_PER_TILE, K), jnp.int32),
        pltpu.VMEM((CH_PER_TILE, K), jnp.int32),
        pltpu.VMEM((K, DH), jnp.float32),
        pltpu.VMEM((K, DH), jnp.float32),
        pltpu.VMEM((K, CW), jnp.float32),
        pltpu.VMEM((NID_K, CW), jnp.float32),
        pltpu.VMEM((ZROWS, DH), jnp.float32),
        pltpu.VMEM((ZROWS, CW), jnp.float32),
        pltpu.VMEM((NID_CHUNKS, NID_K), jnp.int32),
        pltpu.VMEM_SHARED((N_PAD, DH), jnp.float32),
        pltpu.VMEM_SHARED((N_PAD, CW), jnp.float32),
        pltpu.SemaphoreType.DMA,
        pltpu.SemaphoreType.DMA,
    ],
)(_sc_body)


BLK = 1000


def _tc_body(x_ref, feat_ref, cnt_ref, w1_ref, w2a_ref, w2b_ref,
             wi1_ref, wi2a_ref, wi2b_ref, out_ref):
    cnt = (cnt_ref[0, :, 0:1] + cnt_ref[1, :, 0:1]) * 0.5
    idc = cnt_ref[0, :, 1:2] + cnt_ref[1, :, 1:2]
    inv = 1.0 / jnp.maximum(cnt, 1.0)
    ma = feat_ref[0] * inv
    mb = feat_ref[1] * inv
    x = x_ref[...]
    hp = jax.lax.Precision.HIGHEST
    out = (jnp.dot(x, w1_ref[...], precision=hp)
           + jnp.dot(ma, w2a_ref[...], precision=hp)
           + jnp.dot(mb, w2b_ref[...], precision=hp)
           + jnp.dot(idc * x, wi1_ref[...], precision=hp)
           + jnp.dot(idc * ma, wi2a_ref[...], precision=hp)
           + jnp.dot(idc * mb, wi2b_ref[...], precision=hp))
    out_ref[...] = out


_tc_call = pl.pallas_call(
    _tc_body,
    out_shape=jax.ShapeDtypeStruct((N_NODES, D), jnp.float32),
    grid=(N_NODES // BLK,),
    in_specs=[
        pl.BlockSpec((BLK, D), lambda i: (i, 0)),
        pl.BlockSpec((NC, BLK, DH), lambda i: (0, i, 0)),
        pl.BlockSpec((NC, BLK, CW), lambda i: (0, i, 0)),
        pl.BlockSpec((D, D), lambda i: (0, 0)),
        pl.BlockSpec((DH, D), lambda i: (0, 0)),
        pl.BlockSpec((DH, D), lambda i: (0, 0)),
        pl.BlockSpec((D, D), lambda i: (0, 0)),
        pl.BlockSpec((DH, D), lambda i: (0, 0)),
        pl.BlockSpec((DH, D), lambda i: (0, 0)),
    ],
    out_specs=pl.BlockSpec((BLK, D), lambda i: (i, 0)),
)


def kernel(node_feature, edge_index, node_id_index, weight, weight_id):
    x = node_feature
    xh = x.reshape(2 * N_NODES, DH)
    src_r = edge_index[0].reshape(NS, CH_PER_TILE, K)
    src3 = jnp.stack([src_r * 2, src_r * 2 + 1])
    dst3 = edge_index[1].reshape(NS, CH_PER_TILE, K)
    nid2 = node_id_index.reshape(NID_CHUNKS, NID_K)
    col = jax.lax.broadcasted_iota(jnp.int32, (1, CW), 1)
    ones_rows = jnp.broadcast_to((col == 0).astype(jnp.float32), (K, CW))
    idsrc_rows = jnp.broadcast_to((col == 1).astype(jnp.float32), (NID_K, CW))
    feat, cnt = _sc_call(xh, src3, dst3, nid2, ones_rows, idsrc_rows)
    w1 = weight[:D]
    w2a = weight[D:D + DH]
    w2b = weight[D + DH:]
    wi1 = weight_id[:D]
    wi2a = weight_id[D:D + DH]
    wi2b = weight_id[D + DH:]
    return _tc_call(x, feat, cnt, w1, w2a, w2b, wi1, wi2a, wi2b)


# speedup vs baseline: 9.5817x; 1.2238x over previous
import functools

import jax
import jax.numpy as jnp
from jax import lax
from jax.experimental import pallas as pl
from jax.experimental.pallas import tpu as pltpu
from jax.experimental.pallas import tpu_sc as plsc

N_NODES = 10000
N_EDGES = 320000
D = 128
DH = 64
CW = 16
NUM_IDX = 2000

NC = 2
NS = 16
NW = NC * NS
K = 100
CH_PER_TILE = N_EDGES // (NS * K)
N_PAD = 10240
ROWS_PER_TILE = N_PAD // NS
ZROWS = 128
NID_K = 100
NID_CHUNKS = NUM_IDX // NID_K


def _sc_body(xh_hbm, src_hbm, dst_hbm, nid_hbm, ones_hbm, idsrc_hbm,
             feat_hbm, cnt_hbm,
             srcv, dstv, bufa, bufb, onesv, idsrcv, zbuf, zcnt, nidv,
             acc, cac, sema, semb):
    c = lax.axis_index("c")
    s = lax.axis_index("s")

    def zrow(i, carry):
        for jj in range(DH // 16):
            zbuf[i, pl.ds(jj * 16, 16)] = jnp.zeros((16,), jnp.float32)
        zcnt[i, pl.ds(0, 16)] = jnp.zeros((16,), jnp.float32)
        return carry

    lax.fori_loop(0, ZROWS, zrow, 0)
    for r in range(ROWS_PER_TILE // ZROWS):
        base = s * ROWS_PER_TILE + r * ZROWS
        pltpu.sync_copy(zbuf, acc.at[pl.ds(base, ZROWS)])
        pltpu.sync_copy(zcnt, cac.at[pl.ds(base, ZROWS)])
    plsc.subcore_barrier()

    pltpu.sync_copy(src_hbm.at[c, s], srcv)
    pltpu.sync_copy(dst_hbm.at[s], dstv)
    pltpu.sync_copy(ones_hbm, onesv)

    pltpu.async_copy(xh_hbm.at[srcv.at[0]], bufa, sema)
    pltpu.async_copy(xh_hbm.at[srcv.at[1]], bufb, semb)

    def body(it, carry):
        j = it * 2
        do_cnt = (j < CH_PER_TILE // 2) == (c == 0)
        pltpu.make_async_copy(xh_hbm.at[srcv.at[j]], bufa, sema).wait()
        pltpu.sync_copy(bufa, acc.at[dstv.at[j]], add=True)

        @pl.when(do_cnt)
        def _():
            pltpu.sync_copy(onesv, cac.at[dstv.at[j]], add=True)

        @pl.when(j + 2 < CH_PER_TILE)
        def _():
            pltpu.async_copy(xh_hbm.at[srcv.at[j + 2]], bufa, sema)

        pltpu.make_async_copy(xh_hbm.at[srcv.at[j + 1]], bufb, semb).wait()
        pltpu.sync_copy(bufb, acc.at[dstv.at[j + 1]], add=True)

        @pl.when(do_cnt)
        def _():
            pltpu.sync_copy(onesv, cac.at[dstv.at[j + 1]], add=True)

        @pl.when(j + 3 < CH_PER_TILE)
        def _():
            pltpu.async_copy(xh_hbm.at[srcv.at[j + 3]], bufb, semb)

        return carry

    lax.fori_loop(0, CH_PER_TILE // 2, body, 0)

    @pl.when((c == 0) & (s == 0))
    def _():
        pltpu.sync_copy(nid_hbm, nidv)
        pltpu.sync_copy(idsrc_hbm, idsrcv)

        def idbody(j, carry):
            pltpu.sync_copy(idsrcv, cac.at[nidv.at[j]], add=True)
            return carry

        lax.fori_loop(0, NID_CHUNKS, idbody, 0)

    plsc.subcore_barrier()

    rb = s * ROWS_PER_TILE
    pltpu.sync_copy(acc.at[pl.ds(rb, ROWS_PER_TILE)],
                    feat_hbm.at[c, pl.ds(rb, ROWS_PER_TILE)])
    pltpu.sync_copy(cac.at[pl.ds(rb, ROWS_PER_TILE)],
                    cnt_hbm.at[c, pl.ds(rb, ROWS_PER_TILE)])


_sc_call = functools.partial(
    pl.kernel,
    out_type=(jax.ShapeDtypeStruct((NC, N_PAD, DH), jnp.float32),
              jax.ShapeDtypeStruct((NC, N_PAD, CW), jnp.float32)),
    mesh=plsc.VectorSubcoreMesh(core_axis_name="c", subcore_axis_name="s"),
    compiler_params=pltpu.CompilerParams(use_tc_tiling_on_sc=False),
    scratch_types=[
        pltpu.VMEM((CH_PER_TILE, K), jnp.int32),
        pltpu.VMEM((CH_PER_TILE, K), jnp.int32),
        pltpu.VMEM((K, DH), jnp.float32),
        pltpu.VMEM((K, DH), jnp.float32),
        pltpu.VMEM((K, CW), jnp.float32),
        pltpu.VMEM((NID_K, CW), jnp.float32),
        pltpu.VMEM((ZROWS, DH), jnp.float32),
        pltpu.VMEM((ZROWS, CW), jnp.float32),
        pltpu.VMEM((NID_CHUNKS, NID_K), jnp.int32),
        pltpu.VMEM_SHARED((N_PAD, DH), jnp.float32),
        pltpu.VMEM_SHARED((N_PAD, CW), jnp.float32),
        pltpu.SemaphoreType.DMA,
        pltpu.SemaphoreType.DMA,
    ],
)(_sc_body)


BLK = 1000


def _tc_body(x_ref, feat_ref, cnt_ref, w_ref, wid_ref, out_ref):
    cnt = cnt_ref[0, :, 0:1] + cnt_ref[1, :, 0:1]
    idc = cnt_ref[0, :, 1:2] + cnt_ref[1, :, 1:2]
    inv = 1.0 / jnp.maximum(cnt, 1.0)
    aggr = jnp.concatenate(
        [x_ref[...], feat_ref[0] * inv, feat_ref[1] * inv], axis=1)
    hp = jax.lax.Precision.HIGHEST
    out_ref[...] = (jnp.dot(aggr, w_ref[...], precision=hp)
                    + idc * jnp.dot(aggr, wid_ref[...], precision=hp))


_tc_call = pl.pallas_call(
    _tc_body,
    out_shape=jax.ShapeDtypeStruct((N_NODES, D), jnp.float32),
    grid=(N_NODES // BLK,),
    in_specs=[
        pl.BlockSpec((BLK, D), lambda i: (i, 0)),
        pl.BlockSpec((NC, BLK, DH), lambda i: (0, i, 0)),
        pl.BlockSpec((NC, BLK, CW), lambda i: (0, i, 0)),
        pl.BlockSpec((2 * D, D), lambda i: (0, 0)),
        pl.BlockSpec((2 * D, D), lambda i: (0, 0)),
    ],
    out_specs=pl.BlockSpec((BLK, D), lambda i: (i, 0)),
)


def kernel(node_feature, edge_index, node_id_index, weight, weight_id):
    x = node_feature
    xh = x.reshape(2 * N_NODES, DH)
    src_r = edge_index[0].reshape(NS, CH_PER_TILE, K)
    src3 = jnp.stack([src_r * 2, src_r * 2 + 1])
    dst3 = edge_index[1].reshape(NS, CH_PER_TILE, K)
    nid2 = node_id_index.reshape(NID_CHUNKS, NID_K)
    col = jax.lax.broadcasted_iota(jnp.int32, (1, CW), 1)
    ones_rows = jnp.broadcast_to((col == 0).astype(jnp.float32), (K, CW))
    idsrc_rows = jnp.broadcast_to((col == 1).astype(jnp.float32), (NID_K, CW))
    feat, cnt = _sc_call(xh, src3, dst3, nid2, ones_rows, idsrc_rows)
    return _tc_call(x, feat, cnt, weight, weight_id)


# speedup vs baseline: 9.6479x; 1.0069x over previous
import functools

import jax
import jax.numpy as jnp
from jax import lax
from jax.experimental import pallas as pl
from jax.experimental.pallas import tpu as pltpu
from jax.experimental.pallas import tpu_sc as plsc

N_NODES = 10000
N_EDGES = 320000
D = 128
DH = 64
CW = 16
NUM_IDX = 2000

NC = 2
NS = 16
NW = NC * NS
K = 100
CH_PER_TILE = N_EDGES // (NS * K)
N_PAD = 10240
ROWS_PER_TILE = N_PAD // NS
ZROWS = 128
NID_K = 100
NID_CHUNKS = NUM_IDX // NID_K


def _sc_body(xh_hbm, src_hbm, dst_hbm, nid_hbm, ones_hbm, idsrc_hbm,
             feat_hbm, cnt_hbm,
             srcv, dstv, bufa, bufb, onesv, idsrcv, zbuf, zcnt, nidv,
             acc, cac, sema, semb):
    c = lax.axis_index("c")
    s = lax.axis_index("s")

    def zrow(i, carry):
        for jj in range(DH // 16):
            zbuf[i, pl.ds(jj * 16, 16)] = jnp.zeros((16,), jnp.float32)
        zcnt[i, pl.ds(0, 16)] = jnp.zeros((16,), jnp.float32)
        return carry

    lax.fori_loop(0, ZROWS, zrow, 0)
    for r in range(ROWS_PER_TILE // ZROWS):
        base = s * ROWS_PER_TILE + r * ZROWS
        pltpu.sync_copy(zbuf, acc.at[pl.ds(base, ZROWS)])
        pltpu.sync_copy(zcnt, cac.at[pl.ds(base, ZROWS)])
    plsc.subcore_barrier()

    pltpu.sync_copy(src_hbm.at[c, s], srcv)
    pltpu.sync_copy(dst_hbm.at[s], dstv)
    pltpu.sync_copy(ones_hbm, onesv)

    pltpu.async_copy(xh_hbm.at[srcv.at[0]], bufa, sema)
    pltpu.async_copy(xh_hbm.at[srcv.at[1]], bufb, semb)

    def body(it, carry):
        j = it * 2
        do_cnt = (j < CH_PER_TILE // 2) == (c == 0)
        pltpu.make_async_copy(xh_hbm.at[srcv.at[j]], bufa, sema).wait()
        pltpu.sync_copy(bufa, acc.at[dstv.at[j]], add=True)

        @pl.when(do_cnt)
        def _():
            pltpu.sync_copy(onesv, cac.at[dstv.at[j]], add=True)

        @pl.when(j + 2 < CH_PER_TILE)
        def _():
            pltpu.async_copy(xh_hbm.at[srcv.at[j + 2]], bufa, sema)

        pltpu.make_async_copy(xh_hbm.at[srcv.at[j + 1]], bufb, semb).wait()
        pltpu.sync_copy(bufb, acc.at[dstv.at[j + 1]], add=True)

        @pl.when(do_cnt)
        def _():
            pltpu.sync_copy(onesv, cac.at[dstv.at[j + 1]], add=True)

        @pl.when(j + 3 < CH_PER_TILE)
        def _():
            pltpu.async_copy(xh_hbm.at[srcv.at[j + 3]], bufb, semb)

        return carry

    lax.fori_loop(0, CH_PER_TILE // 2, body, 0)

    @pl.when((c == 0) & (s < NID_CHUNKS // 2))
    def _():
        pltpu.sync_copy(nid_hbm, nidv)
        pltpu.sync_copy(idsrc_hbm, idsrcv)

        def idbody(j, carry):
            pltpu.sync_copy(idsrcv, cac.at[nidv.at[j]], add=True)
            return carry

        lax.fori_loop(s * 2, s * 2 + 2, idbody, 0)

    plsc.subcore_barrier()

    rb = s * ROWS_PER_TILE
    pltpu.sync_copy(acc.at[pl.ds(rb, ROWS_PER_TILE)],
                    feat_hbm.at[c, pl.ds(rb, ROWS_PER_TILE)])
    pltpu.sync_copy(cac.at[pl.ds(rb, ROWS_PER_TILE)],
                    cnt_hbm.at[c, pl.ds(rb, ROWS_PER_TILE)])


_sc_call = functools.partial(
    pl.kernel,
    out_type=(jax.ShapeDtypeStruct((NC, N_PAD, DH), jnp.float32),
              jax.ShapeDtypeStruct((NC, N_PAD, CW), jnp.float32)),
    mesh=plsc.VectorSubcoreMesh(core_axis_name="c", subcore_axis_name="s"),
    compiler_params=pltpu.CompilerParams(use_tc_tiling_on_sc=False),
    scratch_types=[
        pltpu.VMEM((CH_PER_TILE, K), jnp.int32),
        pltpu.VMEM((CH_PER_TILE, K), jnp.int32),
        pltpu.VMEM((K, DH), jnp.float32),
        pltpu.VMEM((K, DH), jnp.float32),
        pltpu.VMEM((K, CW), jnp.float32),
        pltpu.VMEM((NID_K, CW), jnp.float32),
        pltpu.VMEM((ZROWS, DH), jnp.float32),
        pltpu.VMEM((ZROWS, CW), jnp.float32),
        pltpu.VMEM((NID_CHUNKS, NID_K), jnp.int32),
        pltpu.VMEM_SHARED((N_PAD, DH), jnp.float32),
        pltpu.VMEM_SHARED((N_PAD, CW), jnp.float32),
        pltpu.SemaphoreType.DMA,
        pltpu.SemaphoreType.DMA,
    ],
)(_sc_body)


BLK = 1000


def _tc_body(x_ref, feat_ref, cnt_ref, w_ref, wid_ref, out_ref):
    cnt = cnt_ref[0, :, 0:1] + cnt_ref[1, :, 0:1]
    idc = cnt_ref[0, :, 1:2] + cnt_ref[1, :, 1:2]
    inv = 1.0 / jnp.maximum(cnt, 1.0)
    aggr = jnp.concatenate(
        [x_ref[...], feat_ref[0] * inv, feat_ref[1] * inv], axis=1)
    hp = jax.lax.Precision.HIGHEST
    out_ref[...] = (jnp.dot(aggr, w_ref[...], precision=hp)
                    + idc * jnp.dot(aggr, wid_ref[...], precision=hp))


_tc_call = pl.pallas_call(
    _tc_body,
    out_shape=jax.ShapeDtypeStruct((N_NODES, D), jnp.float32),
    grid=(N_NODES // BLK,),
    in_specs=[
        pl.BlockSpec((BLK, D), lambda i: (i, 0)),
        pl.BlockSpec((NC, BLK, DH), lambda i: (0, i, 0)),
        pl.BlockSpec((NC, BLK, CW), lambda i: (0, i, 0)),
        pl.BlockSpec((2 * D, D), lambda i: (0, 0)),
        pl.BlockSpec((2 * D, D), lambda i: (0, 0)),
    ],
    out_specs=pl.BlockSpec((BLK, D), lambda i: (i, 0)),
)


def kernel(node_feature, edge_index, node_id_index, weight, weight_id):
    x = node_feature
    xh = x.reshape(2 * N_NODES, DH)
    src_r = edge_index[0].reshape(NS, CH_PER_TILE, K)
    src3 = jnp.stack([src_r * 2, src_r * 2 + 1])
    dst3 = edge_index[1].reshape(NS, CH_PER_TILE, K)
    nid2 = node_id_index.reshape(NID_CHUNKS, NID_K)
    col = jax.lax.broadcasted_iota(jnp.int32, (1, CW), 1)
    ones_rows = jnp.broadcast_to((col == 0).astype(jnp.float32), (K, CW))
    idsrc_rows = jnp.broadcast_to((col == 1).astype(jnp.float32), (NID_K, CW))
    feat, cnt = _sc_call(xh, src3, dst3, nid2, ones_rows, idsrc_rows)
    return _tc_call(x, feat, cnt, weight, weight_id)
